# Initial kernel scaffold; baseline (speedup 1.0000x reference)
import jax
import jax.numpy as jnp
from jax.experimental import pallas as pl


def kernel(x, edge_index, W1a, b1a, W1b, b1b, W2a, b2a, W2b, b2b):
    raise NotImplementedError("write your pallas kernel here")


# speedup vs baseline: 5.4607x; 5.4607x over previous
import functools

import jax
import jax.numpy as jnp
from jax import lax
from jax.experimental import pallas as pl
from jax.experimental.pallas import tpu as pltpu
from jax.experimental.pallas import tpu_sc as plsc

N = 10000
E = 160000
C = 256
HALF = 128
NC = 2
NS = 16
NB = 80
BE = 125
NPAD = 10240
ROWS_PER_SUB = NPAD // NS
ZCHUNK = 120
RB = 1000


def _sc_segment_sum(x2, gidx, didx):
    mesh = plsc.VectorSubcoreMesh(core_axis_name="c", subcore_axis_name="s")

    @functools.partial(
        pl.kernel,
        out_type=jax.ShapeDtypeStruct((NC, NPAD, HALF), jnp.float32),
        mesh=mesh,
        scratch_types=[
            pltpu.VMEM((NB, BE), jnp.int32),
            pltpu.VMEM((NB, BE), jnp.int32),
            pltpu.VMEM((BE, HALF), jnp.float32),
            pltpu.VMEM_SHARED((NPAD, HALF), jnp.float32),
            pltpu.SemaphoreType.DMA,
        ],
    )
    def seg_sum(x2_hbm, gidx_hbm, didx_hbm, out_hbm, sidx, didx, gbuf, acc, sem):
        core = lax.axis_index("c")
        sub = lax.axis_index("s")

        pltpu.sync_copy(gidx_hbm.at[core, sub], sidx)
        pltpu.sync_copy(didx_hbm.at[sub], didx)

        zero = jnp.zeros((16,), jnp.float32)

        @pl.loop(0, BE)
        def _(i):
            for j in range(HALF // 16):
                gbuf[i, pl.ds(j * 16, 16)] = zero

        @pl.loop(0, ROWS_PER_SUB // ZCHUNK)
        def _(i):
            pltpu.sync_copy(
                gbuf.at[pl.ds(0, ZCHUNK)],
                acc.at[pl.ds(sub * ROWS_PER_SUB + i * ZCHUNK, ZCHUNK)],
            )

        pltpu.sync_copy(
            gbuf.at[pl.ds(0, ROWS_PER_SUB % ZCHUNK)],
            acc.at[pl.ds(sub * ROWS_PER_SUB + (ROWS_PER_SUB // ZCHUNK) * ZCHUNK,
                         ROWS_PER_SUB % ZCHUNK)],
        )

        plsc.subcore_barrier()

        @pl.loop(0, NB)
        def _(b):
            pltpu.async_copy(x2_hbm.at[sidx.at[b]], gbuf, sem).wait()
            pltpu.sync_copy(gbuf, acc.at[didx.at[b]], add=True)

        plsc.subcore_barrier()

        pltpu.sync_copy(
            acc.at[pl.ds(sub * ROWS_PER_SUB, ROWS_PER_SUB)],
            out_hbm.at[core, pl.ds(sub * ROWS_PER_SUB, ROWS_PER_SUB)],
        )

    return seg_sum(x2, gidx, didx)


def _tc_mlp(x, a0, a1, Wa, ba, Wb, bb, relu_out):
    def body(x_ref, a0_ref, a1_ref, wa_ref, ba_ref, wb_ref, bb_ref, o_ref):
        h = x_ref[...] + jnp.concatenate([a0_ref[...], a1_ref[...]], axis=1)
        t = jnp.dot(h, wa_ref[...], preferred_element_type=jnp.float32)
        t = jnp.maximum(t + ba_ref[...], 0.0)
        o = jnp.dot(t, wb_ref[...], preferred_element_type=jnp.float32)
        o = o + bb_ref[...]
        if relu_out:
            o = jnp.maximum(o, 0.0)
        o_ref[...] = o

    return pl.pallas_call(
        body,
        grid=(N // RB,),
        in_specs=[
            pl.BlockSpec((RB, C), lambda i: (i, 0)),
            pl.BlockSpec((RB, HALF), lambda i: (i, 0)),
            pl.BlockSpec((RB, HALF), lambda i: (i, 0)),
            pl.BlockSpec((C, C), lambda i: (0, 0)),
            pl.BlockSpec((1, C), lambda i: (0, 0)),
            pl.BlockSpec((C, C), lambda i: (0, 0)),
            pl.BlockSpec((1, C), lambda i: (0, 0)),
        ],
        out_specs=pl.BlockSpec((RB, C), lambda i: (i, 0)),
        out_shape=jax.ShapeDtypeStruct((N, C), jnp.float32),
    )(x, a0, a1, Wa, ba.reshape(1, C), Wb, bb.reshape(1, C))


def kernel(x, edge_index, W1a, b1a, W1b, b1b, W2a, b2a, W2b, b2b):
    src = edge_index[0]
    dst = edge_index[1]
    g0 = src * 2
    gidx = jnp.stack([g0, g0 + 1]).reshape(NC, NS, NB, BE)
    didx = dst.reshape(NS, NB, BE)

    agg1 = _sc_segment_sum(x.reshape(2 * N, HALF), gidx, didx)
    h1 = _tc_mlp(x, agg1[0, :N], agg1[1, :N], W1a, b1a, W1b, b1b, True)
    agg2 = _sc_segment_sum(h1.reshape(2 * N, HALF), gidx, didx)
    out = _tc_mlp(h1, agg2[0, :N], agg2[1, :N], W2a, b2a, W2b, b2b, False)
    return out
